# Initial kernel scaffold; baseline (speedup 1.0000x reference)
import jax
import jax.numpy as jnp
from jax.experimental import pallas as pl


def kernel(x, edge_index, W0, b0, Wconvs, Wout, bout):
    raise NotImplementedError("write your pallas kernel here")


# speedup vs baseline: 3.5583x; 3.5583x over previous
import functools
import math

import jax
import jax.numpy as jnp
from jax import lax
from jax.experimental import pallas as pl
from jax.experimental.pallas import tpu as pltpu
from jax.experimental.pallas import tpu_sc as plsc

N = 10000
E = 10000
M = 320000
D = 128
H = D // 2
NCLS = 40
NLAYERS = 4
ALPHA = 0.1
LAMDA = 0.5

NC, NS = 2, 16
NW = NC * NS
K = 50
ROWS = M // K
TROWS_C = ROWS // NS
TROWS_W = ROWS // NW
CP = 80
NCHUNK = N // CP
CITER = (NCHUNK + NS - 1) // NS

_mesh = plsc.VectorSubcoreMesh(
    core_axis_name="c", subcore_axis_name="s", num_cores=NC, num_subcores=NS
)


def _gather_scatter_body(table2, srcA, srcB, dst2d, out, src_v, dst_v, rows_v,
                         zero_v, acc, sem):
    cid = lax.axis_index("c")
    sid = lax.axis_index("s")

    def zfill(i, _):
        r = i // (H // 16)
        c = i % (H // 16)
        zero_v[r, pl.ds(c * 16, 16)] = jnp.zeros((16,), jnp.float32)
        return 0
    lax.fori_loop(0, CP * (H // 16), zfill, 0)

    def zcopy(k, _):
        q = k * NS + sid
        @pl.when(q < NCHUNK)
        def _():
            pltpu.sync_copy(zero_v, acc.at[pl.ds(q * CP, CP)])
        return 0
    lax.fori_loop(0, CITER, zcopy, 0)
    plsc.subcore_barrier()

    base = sid * TROWS_C

    @pl.when(cid == 0)
    def _():
        pltpu.sync_copy(srcA.at[pl.ds(base, TROWS_C)], src_v)

    @pl.when(cid == 1)
    def _():
        pltpu.sync_copy(srcB.at[pl.ds(base, TROWS_C)], src_v)

    pltpu.sync_copy(dst2d.at[pl.ds(base, TROWS_C)], dst_v)

    def step(j, _):
        pltpu.async_copy(table2.at[src_v.at[j]], rows_v, sem).wait()
        pltpu.sync_copy(rows_v, acc.at[dst_v.at[j]], add=True)
        return 0
    lax.fori_loop(0, TROWS_C, step, 0)
    plsc.subcore_barrier()

    def wback(k, _):
        q = k * NS + sid
        @pl.when(q < NCHUNK)
        def _():
            r0 = q * CP
            pltpu.sync_copy(acc.at[pl.ds(r0, CP)],
                            out.at[pl.ds(cid * N + r0, CP)])
        return 0
    lax.fori_loop(0, CITER, wback, 0)


_sc_gather_scatter = pl.kernel(
    _gather_scatter_body,
    out_type=[jax.ShapeDtypeStruct((NC * N, H), jnp.float32)],
    mesh=_mesh,
    compiler_params=pltpu.CompilerParams(use_tc_tiling_on_sc=False),
    scratch_types=[
        pltpu.VMEM((TROWS_C, K), jnp.int32),
        pltpu.VMEM((TROWS_C, K), jnp.int32),
        pltpu.VMEM((K, H), jnp.float32),
        pltpu.VMEM((CP, H), jnp.float32),
        pltpu.VMEM_SHARED((N, H), jnp.float32),
        pltpu.SemaphoreType.DMA,
    ],
)


def _counts_body(v2d, e2d, outv, oute, vidx, eidx, ones_v, zero_v, accv, acce):
    cid = lax.axis_index("c")
    sid = lax.axis_index("s")
    wid = sid * NC + cid

    def ofill(r, _):
        ones_v[r, :] = jnp.ones((16,), jnp.float32)
        return 0
    lax.fori_loop(0, K, ofill, 0)

    def zfill(r, _):
        zero_v[r, :] = jnp.zeros((16,), jnp.float32)
        return 0
    lax.fori_loop(0, CP, zfill, 0)

    def zcopy(k, _):
        q = k * NS + sid
        @pl.when(q < NCHUNK)
        def _():
            pltpu.sync_copy(zero_v, accv.at[pl.ds(q * CP, CP)])
            pltpu.sync_copy(zero_v, acce.at[pl.ds(q * CP, CP)])
        return 0
    lax.fori_loop(0, CITER, zcopy, 0)
    plsc.subcore_barrier()

    base = wid * TROWS_W
    pltpu.sync_copy(v2d.at[pl.ds(base, TROWS_W)], vidx)
    pltpu.sync_copy(e2d.at[pl.ds(base, TROWS_W)], eidx)

    def step(j, _):
        pltpu.sync_copy(ones_v, accv.at[vidx.at[j]], add=True)
        pltpu.sync_copy(ones_v, acce.at[eidx.at[j]], add=True)
        return 0
    lax.fori_loop(0, TROWS_W, step, 0)
    plsc.subcore_barrier()

    def wback(k, _):
        q = k * NS + sid
        @pl.when(q < NCHUNK)
        def _():
            r0 = q * CP
            pltpu.sync_copy(accv.at[pl.ds(r0, CP)],
                            outv.at[pl.ds(cid * N + r0, CP)])
            pltpu.sync_copy(acce.at[pl.ds(r0, CP)],
                            oute.at[pl.ds(cid * N + r0, CP)])
        return 0
    lax.fori_loop(0, CITER, wback, 0)


_sc_counts = pl.kernel(
    _counts_body,
    out_type=[
        jax.ShapeDtypeStruct((NC * N, 16), jnp.float32),
        jax.ShapeDtypeStruct((NC * E, 16), jnp.float32),
    ],
    mesh=_mesh,
    compiler_params=pltpu.CompilerParams(use_tc_tiling_on_sc=False),
    scratch_types=[
        pltpu.VMEM((TROWS_W, K), jnp.int32),
        pltpu.VMEM((TROWS_W, K), jnp.int32),
        pltpu.VMEM((K, 16), jnp.float32),
        pltpu.VMEM((CP, 16), jnp.float32),
        pltpu.VMEM_SHARED((N, 16), jnp.float32),
        pltpu.VMEM_SHARED((E, 16), jnp.float32),
    ],
)


_BR = 1000


def _lin_relu_body(x_ref, w_ref, b_ref, o_ref):
    acc = lax.dot_general(x_ref[...], w_ref[...], (((1,), (1,)), ((), ())),
                          preferred_element_type=jnp.float32)
    o_ref[...] = jnp.maximum(acc + b_ref[...], 0.0)


def _tc_linear_relu(x, W, b):
    return pl.pallas_call(
        _lin_relu_body,
        grid=(N // _BR,),
        in_specs=[
            pl.BlockSpec((_BR, D), lambda i: (i, 0)),
            pl.BlockSpec((D, D), lambda i: (0, 0)),
            pl.BlockSpec((1, D), lambda i: (0, 0)),
        ],
        out_specs=pl.BlockSpec((_BR, D), lambda i: (i, 0)),
        out_shape=jax.ShapeDtypeStruct((N, D), jnp.float32),
    )(x, W, b)


def _combine_body(pl_ref, pr_ref, c0_ref, c1_ref, o_ref):
    cnt = jnp.maximum(c0_ref[:, 0:1] + c1_ref[:, 0:1], 1.0)
    s = jnp.concatenate([pl_ref[...], pr_ref[...]], axis=1)
    o_ref[...] = s / cnt


def _tc_combine(p, c):
    return pl.pallas_call(
        _combine_body,
        grid=(N // _BR,),
        in_specs=[
            pl.BlockSpec((_BR, H), lambda i: (i, 0)),
            pl.BlockSpec((_BR, H), lambda i: (i + N // _BR, 0)),
            pl.BlockSpec((_BR, 16), lambda i: (i, 0)),
            pl.BlockSpec((_BR, 16), lambda i: (i + N // _BR, 0)),
        ],
        out_specs=pl.BlockSpec((_BR, D), lambda i: (i, 0)),
        out_shape=jax.ShapeDtypeStruct((N, D), jnp.float32),
    )(p, p, c, c)


def _layer_body(ql_ref, qr_ref, c0_ref, c1_ref, h0_ref, w_ref, o_ref, *, beta):
    cnt = jnp.maximum(c0_ref[:, 0:1] + c1_ref[:, 0:1], 1.0)
    xv = jnp.concatenate([ql_ref[...], qr_ref[...]], axis=1) / cnt
    xi = (1.0 - ALPHA) * xv + ALPHA * h0_ref[...]
    mm = lax.dot_general(xi, w_ref[...], (((1,), (1,)), ((), ())),
                         preferred_element_type=jnp.float32)
    o_ref[...] = jnp.maximum((1.0 - beta) * xi + beta * mm, 0.0)


def _tc_layer(q, c, h0, W, beta):
    return pl.pallas_call(
        functools.partial(_layer_body, beta=beta),
        grid=(N // _BR,),
        in_specs=[
            pl.BlockSpec((_BR, H), lambda i: (i, 0)),
            pl.BlockSpec((_BR, H), lambda i: (i + N // _BR, 0)),
            pl.BlockSpec((_BR, 16), lambda i: (i, 0)),
            pl.BlockSpec((_BR, 16), lambda i: (i + N // _BR, 0)),
            pl.BlockSpec((_BR, D), lambda i: (i, 0)),
            pl.BlockSpec((D, D), lambda i: (0, 0)),
        ],
        out_specs=pl.BlockSpec((_BR, D), lambda i: (i, 0)),
        out_shape=jax.ShapeDtypeStruct((N, D), jnp.float32),
    )(q, q, c, c, h0, W)


def _out_body(h_ref, w_ref, b_ref, o_ref):
    acc = lax.dot_general(h_ref[...], w_ref[...], (((1,), (1,)), ((), ())),
                          preferred_element_type=jnp.float32)
    o_ref[...] = acc + b_ref[...]


def _tc_out(h, W, b):
    return pl.pallas_call(
        _out_body,
        grid=(N // _BR,),
        in_specs=[
            pl.BlockSpec((_BR, D), lambda i: (i, 0)),
            pl.BlockSpec((NCLS, D), lambda i: (0, 0)),
            pl.BlockSpec((1, NCLS), lambda i: (0, 0)),
        ],
        out_specs=pl.BlockSpec((_BR, NCLS), lambda i: (i, 0)),
        out_shape=jax.ShapeDtypeStruct((N, NCLS), jnp.float32),
    )(h, W, b)


def kernel(x, edge_index, W0, b0, Wconvs, Wout, bout):
    vertex2d = edge_index[0].reshape(ROWS, K)
    edges2d = edge_index[1].reshape(ROWS, K)
    vA = vertex2d * 2
    vB = vA + 1
    eA = edges2d * 2
    eB = eA + 1

    cntv, cnte = _sc_counts(vertex2d, edges2d)

    h = _tc_linear_relu(x, W0, b0.reshape(1, D))
    h0 = h
    for i in range(NLAYERS):
        beta = math.log(LAMDA / (i + 1) + 1.0)
        pe = _sc_gather_scatter(h.reshape(NC * N, H), vA, vB, edges2d)[0]
        xe = _tc_combine(pe, cnte)
        pv = _sc_gather_scatter(xe.reshape(NC * N, H), eA, eB, vertex2d)[0]
        h = _tc_layer(pv, cntv, h0, Wconvs[i], beta)

    return _tc_out(h, Wout, bout.reshape(1, NCLS))


# speedup vs baseline: 11.2231x; 3.1541x over previous
import functools
import math

import jax
import jax.numpy as jnp
from jax import lax
from jax.experimental import pallas as pl
from jax.experimental.pallas import tpu as pltpu
from jax.experimental.pallas import tpu_sc as plsc

N = 10000
E = 10000
M = 320000
D = 128
H = D // 2
NCLS = 40
NLAYERS = 4
ALPHA = 0.1
LAMDA = 0.5

NC, NS = 2, 16
NW = NC * NS
K = 100
ROWS = M // K
TROWS_C = ROWS // NS
NB = 4
CB = TROWS_C // NB
KW = 50
ROWS_W = M // KW
TROWS_W = ROWS_W // NW
CP = 80
NCHUNK = N // CP
CITER = (NCHUNK + NS - 1) // NS

_mesh = plsc.VectorSubcoreMesh(
    core_axis_name="c", subcore_axis_name="s", num_cores=NC, num_subcores=NS
)


def _gather_scatter_body(table2, srcA, srcB, dst2d, out, src_v, dst_v,
                         rows0, rows1, rows2, rows3,
                         zero_v, acc, sem0, sem1, sem2, sem3):
    bufs = (rows0, rows1, rows2, rows3)
    sems = (sem0, sem1, sem2, sem3)
    cid = lax.axis_index("c")
    sid = lax.axis_index("s")

    def zfill(i, _):
        r = i // (H // 16)
        c = i % (H // 16)
        zero_v[r, pl.ds(c * 16, 16)] = jnp.zeros((16,), jnp.float32)
        return 0
    lax.fori_loop(0, CP * (H // 16), zfill, 0)

    def zcopy(k, _):
        q = k * NS + sid
        @pl.when(q < NCHUNK)
        def _():
            pltpu.sync_copy(zero_v, acc.at[pl.ds(q * CP, CP)])
        return 0
    lax.fori_loop(0, CITER, zcopy, 0)
    plsc.subcore_barrier()

    base = sid * TROWS_C

    @pl.when(cid == 0)
    def _():
        pltpu.sync_copy(srcA.at[pl.ds(base, TROWS_C)], src_v)

    @pl.when(cid == 1)
    def _():
        pltpu.sync_copy(srcB.at[pl.ds(base, TROWS_C)], src_v)

    pltpu.sync_copy(dst2d.at[pl.ds(base, TROWS_C)], dst_v)

    for b in range(NB):
        pltpu.async_copy(table2.at[src_v.at[b]], bufs[b], sems[b])

    def outer(g, _):
        for b in range(NB):
            j = g * NB + b
            pltpu.make_async_copy(table2.at[src_v.at[j]], bufs[b],
                                  sems[b]).wait()
            pltpu.sync_copy(bufs[b], acc.at[dst_v.at[j]], add=True)
            pltpu.async_copy(table2.at[src_v.at[j + NB]], bufs[b], sems[b])
        return 0
    lax.fori_loop(0, CB - 1, outer, 0)

    for b in range(NB):
        j = (CB - 1) * NB + b
        pltpu.make_async_copy(table2.at[src_v.at[j]], bufs[b], sems[b]).wait()
        pltpu.sync_copy(bufs[b], acc.at[dst_v.at[j]], add=True)
    plsc.subcore_barrier()

    def wback(k, _):
        q = k * NS + sid
        @pl.when(q < NCHUNK)
        def _():
            r0 = q * CP
            pltpu.sync_copy(acc.at[pl.ds(r0, CP)],
                            out.at[pl.ds(cid * N + r0, CP)])
        return 0
    lax.fori_loop(0, CITER, wback, 0)


_sc_gather_scatter = pl.kernel(
    _gather_scatter_body,
    out_type=[jax.ShapeDtypeStruct((NC * N, H), jnp.float32)],
    mesh=_mesh,
    compiler_params=pltpu.CompilerParams(use_tc_tiling_on_sc=False),
    scratch_types=[
        pltpu.VMEM((TROWS_C, K), jnp.int32),
        pltpu.VMEM((TROWS_C, K), jnp.int32),
        pltpu.VMEM((K, H), jnp.float32),
        pltpu.VMEM((K, H), jnp.float32),
        pltpu.VMEM((K, H), jnp.float32),
        pltpu.VMEM((K, H), jnp.float32),
        pltpu.VMEM((CP, H), jnp.float32),
        pltpu.VMEM_SHARED((N, H), jnp.float32),
        pltpu.SemaphoreType.DMA,
        pltpu.SemaphoreType.DMA,
        pltpu.SemaphoreType.DMA,
        pltpu.SemaphoreType.DMA,
    ],
)


def _counts_body(v2d, e2d, outv, oute, vidx, eidx, ones_v, zero_v, accv, acce):
    cid = lax.axis_index("c")
    sid = lax.axis_index("s")
    wid = sid * NC + cid

    def ofill(r, _):
        ones_v[r, :] = jnp.ones((16,), jnp.float32)
        return 0
    lax.fori_loop(0, KW, ofill, 0)

    def zfill(r, _):
        zero_v[r, :] = jnp.zeros((16,), jnp.float32)
        return 0
    lax.fori_loop(0, CP, zfill, 0)

    def zcopy(k, _):
        q = k * NS + sid
        @pl.when(q < NCHUNK)
        def _():
            pltpu.sync_copy(zero_v, accv.at[pl.ds(q * CP, CP)])
            pltpu.sync_copy(zero_v, acce.at[pl.ds(q * CP, CP)])
        return 0
    lax.fori_loop(0, CITER, zcopy, 0)
    plsc.subcore_barrier()

    base = wid * TROWS_W
    pltpu.sync_copy(v2d.at[pl.ds(base, TROWS_W)], vidx)
    pltpu.sync_copy(e2d.at[pl.ds(base, TROWS_W)], eidx)

    def step(j, _):
        pltpu.sync_copy(ones_v, accv.at[vidx.at[j]], add=True)
        pltpu.sync_copy(ones_v, acce.at[eidx.at[j]], add=True)
        return 0
    lax.fori_loop(0, TROWS_W, step, 0)
    plsc.subcore_barrier()

    def wback(k, _):
        q = k * NS + sid
        @pl.when(q < NCHUNK)
        def _():
            r0 = q * CP
            pltpu.sync_copy(accv.at[pl.ds(r0, CP)],
                            outv.at[pl.ds(cid * N + r0, CP)])
            pltpu.sync_copy(acce.at[pl.ds(r0, CP)],
                            oute.at[pl.ds(cid * N + r0, CP)])
        return 0
    lax.fori_loop(0, CITER, wback, 0)


_sc_counts = pl.kernel(
    _counts_body,
    out_type=[
        jax.ShapeDtypeStruct((NC * N, 16), jnp.float32),
        jax.ShapeDtypeStruct((NC * E, 16), jnp.float32),
    ],
    mesh=_mesh,
    compiler_params=pltpu.CompilerParams(use_tc_tiling_on_sc=False),
    scratch_types=[
        pltpu.VMEM((TROWS_W, KW), jnp.int32),
        pltpu.VMEM((TROWS_W, KW), jnp.int32),
        pltpu.VMEM((KW, 16), jnp.float32),
        pltpu.VMEM((CP, 16), jnp.float32),
        pltpu.VMEM_SHARED((N, 16), jnp.float32),
        pltpu.VMEM_SHARED((E, 16), jnp.float32),
    ],
)


_BR = 1000


def _lin_relu_body(x_ref, w_ref, b_ref, o_ref):
    acc = lax.dot_general(x_ref[...], w_ref[...], (((1,), (1,)), ((), ())),
                          preferred_element_type=jnp.float32)
    o_ref[...] = jnp.maximum(acc + b_ref[...], 0.0)


def _tc_linear_relu(x, W, b):
    return pl.pallas_call(
        _lin_relu_body,
        grid=(N // _BR,),
        in_specs=[
            pl.BlockSpec((_BR, D), lambda i: (i, 0)),
            pl.BlockSpec((D, D), lambda i: (0, 0)),
            pl.BlockSpec((1, D), lambda i: (0, 0)),
        ],
        out_specs=pl.BlockSpec((_BR, D), lambda i: (i, 0)),
        out_shape=jax.ShapeDtypeStruct((N, D), jnp.float32),
    )(x, W, b)


def _combine_body(pl_ref, pr_ref, c0_ref, c1_ref, o_ref):
    cnt = jnp.maximum(c0_ref[:, 0:1] + c1_ref[:, 0:1], 1.0)
    s = jnp.concatenate([pl_ref[...], pr_ref[...]], axis=1)
    o_ref[...] = s / cnt


def _tc_combine(p, c):
    return pl.pallas_call(
        _combine_body,
        grid=(N // _BR,),
        in_specs=[
            pl.BlockSpec((_BR, H), lambda i: (i, 0)),
            pl.BlockSpec((_BR, H), lambda i: (i + N // _BR, 0)),
            pl.BlockSpec((_BR, 16), lambda i: (i, 0)),
            pl.BlockSpec((_BR, 16), lambda i: (i + N // _BR, 0)),
        ],
        out_specs=pl.BlockSpec((_BR, D), lambda i: (i, 0)),
        out_shape=jax.ShapeDtypeStruct((N, D), jnp.float32),
    )(p, p, c, c)


def _layer_body(ql_ref, qr_ref, c0_ref, c1_ref, h0_ref, w_ref, o_ref, *, beta):
    cnt = jnp.maximum(c0_ref[:, 0:1] + c1_ref[:, 0:1], 1.0)
    xv = jnp.concatenate([ql_ref[...], qr_ref[...]], axis=1) / cnt
    xi = (1.0 - ALPHA) * xv + ALPHA * h0_ref[...]
    mm = lax.dot_general(xi, w_ref[...], (((1,), (1,)), ((), ())),
                         preferred_element_type=jnp.float32)
    o_ref[...] = jnp.maximum((1.0 - beta) * xi + beta * mm, 0.0)


def _tc_layer(q, c, h0, W, beta):
    return pl.pallas_call(
        functools.partial(_layer_body, beta=beta),
        grid=(N // _BR,),
        in_specs=[
            pl.BlockSpec((_BR, H), lambda i: (i, 0)),
            pl.BlockSpec((_BR, H), lambda i: (i + N // _BR, 0)),
            pl.BlockSpec((_BR, 16), lambda i: (i, 0)),
            pl.BlockSpec((_BR, 16), lambda i: (i + N // _BR, 0)),
            pl.BlockSpec((_BR, D), lambda i: (i, 0)),
            pl.BlockSpec((D, D), lambda i: (0, 0)),
        ],
        out_specs=pl.BlockSpec((_BR, D), lambda i: (i, 0)),
        out_shape=jax.ShapeDtypeStruct((N, D), jnp.float32),
    )(q, q, c, c, h0, W)


def _out_body(h_ref, w_ref, b_ref, o_ref):
    acc = lax.dot_general(h_ref[...], w_ref[...], (((1,), (1,)), ((), ())),
                          preferred_element_type=jnp.float32)
    o_ref[...] = acc + b_ref[...]


def _tc_out(h, W, b):
    return pl.pallas_call(
        _out_body,
        grid=(N // _BR,),
        in_specs=[
            pl.BlockSpec((_BR, D), lambda i: (i, 0)),
            pl.BlockSpec((NCLS, D), lambda i: (0, 0)),
            pl.BlockSpec((1, NCLS), lambda i: (0, 0)),
        ],
        out_specs=pl.BlockSpec((_BR, NCLS), lambda i: (i, 0)),
        out_shape=jax.ShapeDtypeStruct((N, NCLS), jnp.float32),
    )(h, W, b)


def kernel(x, edge_index, W0, b0, Wconvs, Wout, bout):
    vertex2d = edge_index[0].reshape(ROWS, K)
    edges2d = edge_index[1].reshape(ROWS, K)
    vertexw = edge_index[0].reshape(ROWS_W, KW)
    edgesw = edge_index[1].reshape(ROWS_W, KW)
    vA = vertex2d * 2
    vB = vA + 1
    eA = edges2d * 2
    eB = eA + 1

    cntv, cnte = _sc_counts(vertexw, edgesw)

    h = _tc_linear_relu(x, W0, b0.reshape(1, D))
    h0 = h
    for i in range(NLAYERS):
        beta = math.log(LAMDA / (i + 1) + 1.0)
        pe = _sc_gather_scatter(h.reshape(NC * N, H), vA, vB, edges2d)[0]
        xe = _tc_combine(pe, cnte)
        pv = _sc_gather_scatter(xe.reshape(NC * N, H), eA, eB, vertex2d)[0]
        h = _tc_layer(pv, cntv, h0, Wconvs[i], beta)

    return _tc_out(h, Wout, bout.reshape(1, NCLS))
